# baseline (speedup 1.0000x reference)
import functools

import jax
import jax.numpy as jnp
from jax import lax
from jax.experimental import pallas as pl
from jax.experimental.pallas import tpu as pltpu
from jax.experimental.pallas import tpu_sc as plsc

_N = 1024
_NC, _NS, _L = 2, 16, 16
_NW = _NC * _NS
_ROWS_PER_W = _N // _NW
_GROUPS = _ROWS_PER_W // _L


def _dist_kernel(emb_ref, dist_ref):
    n, _ = emb_ref.shape
    emb = emb_ref[...]
    sq = jnp.sum(emb * emb, axis=1, keepdims=True)
    en = emb * jax.lax.rsqrt(jnp.maximum(sq, 1e-30))
    g = jax.lax.dot_general(en, en, (((1,), (1,)), ((), ())),
                            preferred_element_type=jnp.float32,
                            precision=jax.lax.Precision.HIGHEST)
    row_ids = jax.lax.broadcasted_iota(jnp.int32, (n, n), 0)
    col_ids = jax.lax.broadcasted_iota(jnp.int32, (n, n), 1)
    dist_ref[...] = jnp.where(row_ids == col_ids, -jnp.inf, 1.0 - g)


def _sc_reduce_body(dist_hbm, lab_hbm, pos_d_hbm, neg_d_hbm,
                    pos_i_hbm, neg_i_hbm, buf, lab_v, spd, snd, spi, sni):
    c = lax.axis_index("c")
    s = lax.axis_index("s")
    wid = s * _NC + c
    base = wid * _ROWS_PER_W

    pltpu.sync_copy(lab_hbm, lab_v)
    lane = lax.broadcasted_iota(jnp.int32, (_L,), 0)
    inf = jnp.float32(jnp.inf)

    for t in range(_GROUPS):
        gbase = base + t * _L
        pltpu.sync_copy(dist_hbm.at[pl.ds(gbase, _L)], buf)
        lab16 = plsc.load_gather(lab_v, [gbase + lane])

        def col_body(j, ch):
            bpv, bpi, bnv, bni = ch
            jv = jnp.full((_L,), j, jnp.int32)
            dvec = plsc.load_gather(buf, [lane, jv])
            labc = plsc.load_gather(lab_v, [jv])
            jf = j.astype(jnp.float32)
            m = labc == lab16
            pv = jnp.where(m, dvec, -inf)
            bpi = jnp.where(pv > bpv, jf, bpi)
            bpv = jnp.maximum(bpv, pv)
            nv = jnp.where(m, inf, dvec)
            bni = jnp.where(nv < bnv, jf, bni)
            bnv = jnp.minimum(bnv, nv)
            return (bpv, bpi, bnv, bni)

        init = (jnp.full((_L,), -inf), jnp.zeros((_L,), jnp.float32),
                jnp.full((_L,), inf), jnp.zeros((_L,), jnp.float32))
        bpv, bpi, bnv, bni = lax.fori_loop(0, _N, col_body, init)

        spd[pl.ds(t * _L, _L)] = bpv
        spi[pl.ds(t * _L, _L)] = bpi
        snd[pl.ds(t * _L, _L)] = bnv
        sni[pl.ds(t * _L, _L)] = bni

    pltpu.sync_copy(spd, pos_d_hbm.at[pl.ds(base, _ROWS_PER_W)])
    pltpu.sync_copy(snd, neg_d_hbm.at[pl.ds(base, _ROWS_PER_W)])
    pltpu.sync_copy(spi, pos_i_hbm.at[pl.ds(base, _ROWS_PER_W)])
    pltpu.sync_copy(sni, neg_i_hbm.at[pl.ds(base, _ROWS_PER_W)])


def kernel(embeddings, labels):
    n, d = embeddings.shape

    dist = pl.pallas_call(
        _dist_kernel,
        out_shape=jax.ShapeDtypeStruct((n, n), jnp.float32),
    )(embeddings)

    sc_reduce = functools.partial(
        pl.kernel,
        out_type=(
            jax.ShapeDtypeStruct((n,), jnp.float32),
            jax.ShapeDtypeStruct((n,), jnp.float32),
            jax.ShapeDtypeStruct((n,), jnp.float32),
            jax.ShapeDtypeStruct((n,), jnp.float32),
        ),
        mesh=plsc.VectorSubcoreMesh(core_axis_name="c", subcore_axis_name="s"),
        scratch_types=[
            pltpu.VMEM((_L, n), jnp.float32),
            pltpu.VMEM((n,), jnp.float32),
            pltpu.VMEM((_ROWS_PER_W,), jnp.float32),
            pltpu.VMEM((_ROWS_PER_W,), jnp.float32),
            pltpu.VMEM((_ROWS_PER_W,), jnp.float32),
            pltpu.VMEM((_ROWS_PER_W,), jnp.float32),
        ],
        compiler_params=pltpu.CompilerParams(needs_layout_passes=False),
    )(_sc_reduce_body)

    pos_d, neg_d, pos_if, neg_if = sc_reduce(dist, labels.astype(jnp.float32))

    anchors = jnp.arange(n, dtype=jnp.int32)
    triplets = jnp.column_stack((anchors, pos_if.astype(jnp.int32),
                                 neg_if.astype(jnp.int32)))
    return (triplets, pos_d, neg_d)
